# baseline (speedup 1.0000x reference)
import functools

import jax
import jax.numpy as jnp
from jax import lax
from jax.experimental import pallas as pl
from jax.experimental.pallas import tpu as pltpu
from jax.experimental.pallas import tpu_sc as plsc

N = 10000
D = 128
E = 320000
HID = 128

NC = 2
NS = 16
NW = NC * NS
BPW = 128
RPS = 632
NP = NS * RPS

NBLK_F = 80
EPW_F = NBLK_F * BPW
EPAD = NW * EPW_F


def _agg_body(x_hbm, row_hbm, col_hbm, init_hbm, out_hbm,
              idxr_v, idxc_v, rows_v, agg_sh, sem):
  c = lax.axis_index("c")
  s = lax.axis_index("s")
  w = s * NC + c
  base = w * EPW_F
  zbase = s * RPS

  pltpu.sync_copy(init_hbm.at[pl.ds(c * NP + zbase, RPS)],
                  agg_sh.at[pl.ds(zbase, RPS)])
  plsc.subcore_barrier()

  @pl.loop(0, NBLK_F)
  def _blocks(j):
    off = base + j * BPW
    pltpu.sync_copy(row_hbm.at[pl.ds(off, BPW)], idxr_v)
    pltpu.sync_copy(col_hbm.at[pl.ds(off, BPW)], idxc_v)
    pltpu.async_copy(x_hbm.at[idxc_v], rows_v, sem).wait()
    pltpu.sync_copy(rows_v, agg_sh.at[idxr_v], add=True)

  plsc.subcore_barrier()
  pltpu.sync_copy(agg_sh.at[pl.ds(zbase, RPS)],
                  out_hbm.at[pl.ds(c * NP + zbase, RPS)])


_agg_pass = functools.partial(
    pl.kernel,
    out_type=jax.ShapeDtypeStruct((NC * NP, D), jnp.float32),
    mesh=plsc.VectorSubcoreMesh(core_axis_name="c", subcore_axis_name="s"),
    scratch_types=[
        pltpu.VMEM((BPW,), jnp.int32),
        pltpu.VMEM((BPW,), jnp.int32),
        pltpu.VMEM((BPW, D), jnp.float32),
        pltpu.VMEM_SHARED((NP, D), jnp.float32),
        pltpu.SemaphoreType.DMA,
    ],
)(_agg_body)


def _deg_body(row_hbm, zeros_hbm, ones_hbm, out_hbm,
              idxr_v, ones_v, deg_sh, sem):
  c = lax.axis_index("c")
  s = lax.axis_index("s")
  w = s * NC + c
  base = w * EPW_F
  zbase = s * RPS

  pltpu.sync_copy(ones_hbm, ones_v)
  pltpu.sync_copy(zeros_hbm.at[pl.ds(zbase, RPS)],
                  deg_sh.at[pl.ds(zbase, RPS)])
  plsc.subcore_barrier()

  @pl.loop(0, NBLK_F)
  def _blocks(j):
    pltpu.sync_copy(row_hbm.at[pl.ds(base + j * BPW, BPW)], idxr_v)
    pltpu.sync_copy(ones_v, deg_sh.at[idxr_v], add=True)

  plsc.subcore_barrier()
  pltpu.sync_copy(deg_sh.at[pl.ds(zbase, RPS)],
                  out_hbm.at[pl.ds(c * NP + zbase, RPS)])


_deg_pass = functools.partial(
    pl.kernel,
    out_type=jax.ShapeDtypeStruct((NC * NP, D), jnp.float32),
    mesh=plsc.VectorSubcoreMesh(core_axis_name="c", subcore_axis_name="s"),
    scratch_types=[
        pltpu.VMEM((BPW,), jnp.int32),
        pltpu.VMEM((BPW, D), jnp.float32),
        pltpu.VMEM_SHARED((NP, D), jnp.float32),
        pltpu.SemaphoreType.DMA,
    ],
)(_deg_body)


ROWS_TC = 1000
NTCBLK = N // ROWS_TC


def _mlp_body(x_ref, agg2_ref, deg2_ref, w1x, w1a, b1, w2, b2, w3, b3, w4,
              b4, out_ref):
  deg = deg2_ref[0, :, 0:1] + deg2_ref[1, :, 0:1]
  deg = jnp.maximum(deg, 1.0)
  agg = (agg2_ref[0] + agg2_ref[1]) / deg
  h = jnp.dot(x_ref[...], w1x[...], preferred_element_type=jnp.float32)
  h += jnp.dot(agg, w1a[...], preferred_element_type=jnp.float32)
  h = jnp.maximum(h + b1[...], 0.0)
  h = jnp.maximum(
      jnp.dot(h, w2[...], preferred_element_type=jnp.float32) + b2[...], 0.0)
  h = jnp.maximum(
      jnp.dot(h, w3[...], preferred_element_type=jnp.float32) + b3[...], 0.0)
  out_ref[...] = (
      jnp.dot(h, w4[...], preferred_element_type=jnp.float32) + b4[...])


def _full_spec(shape):
  return pl.BlockSpec(shape, lambda i: tuple(0 for _ in shape))


_mlp = pl.pallas_call(
    _mlp_body,
    grid=(NTCBLK,),
    in_specs=[
        pl.BlockSpec((ROWS_TC, D), lambda i: (i, 0)),
        pl.BlockSpec((NC, ROWS_TC, D), lambda i: (0, i, 0)),
        pl.BlockSpec((NC, ROWS_TC, D), lambda i: (0, i, 0)),
        _full_spec((D, HID)),
        _full_spec((D, HID)),
        _full_spec((1, HID)),
        _full_spec((HID, HID)),
        _full_spec((1, HID)),
        _full_spec((HID, HID)),
        _full_spec((1, HID)),
        _full_spec((HID, D)),
        _full_spec((1, D)),
    ],
    out_specs=pl.BlockSpec((ROWS_TC, D), lambda i: (i, 0)),
    out_shape=jax.ShapeDtypeStruct((N, D), jnp.float32),
)


@jax.jit
def kernel(x, edge_index, W1, b1, W2, b2, W3, b3, W4, b4):
  row = edge_index[0].astype(jnp.int32)
  col = edge_index[1].astype(jnp.int32)
  pad = EPAD - E
  row_p = jnp.concatenate([row, jnp.full((pad,), N, jnp.int32)])
  col_p = jnp.concatenate([col, jnp.zeros((pad,), jnp.int32)])
  zeros_slab = jnp.zeros((NC * NP, D), jnp.float32)

  part = _agg_pass(x, row_p, col_p, zeros_slab)
  deg128 = _deg_pass(row_p, zeros_slab, jnp.ones((BPW, D), jnp.float32))

  agg2 = part.reshape(NC, NP, D)
  deg2 = deg128.reshape(NC, NP, D)

  w1t = W1.T
  return _mlp(x, agg2, deg2, w1t[:D], w1t[D:], b1.reshape(1, HID),
              W2.T, b2.reshape(1, HID), W3.T, b3.reshape(1, HID),
              W4.T, b4.reshape(1, D))


# speedup vs baseline: 1.3671x; 1.3671x over previous
import functools

import jax
import jax.numpy as jnp
from jax import lax
from jax.experimental import pallas as pl
from jax.experimental.pallas import tpu as pltpu
from jax.experimental.pallas import tpu_sc as plsc

N = 10000
D = 128
E = 320000
HID = 128

NC = 2
NS = 16
NW = NC * NS
BPW = 128
RPS = 632
NP = NS * RPS

NBLK = 80
EPW = NBLK * BPW
EPAD = NW * EPW
NBLK_H = 40
EPW_H = NBLK_H * BPW
EH = NW * EPW_H

NROWB = 2
NIDXB = 8


def _agg_body(x_hbm, row_hbm, col_hbm, init_hbm, out_hbm,
              idxr_v, idxc_v, rows_v, agg_sh, sem_i, sem_g, sem_s):
  c = lax.axis_index("c")
  s = lax.axis_index("s")
  w = s * NC + c
  base = w * EPW_H
  zbase = s * RPS

  def fire_idx(k, blk):
    off = base + blk * BPW
    pltpu.async_copy(row_hbm.at[pl.ds(off, BPW)], idxr_v.at[k], sem_i.at[k])
    pltpu.async_copy(col_hbm.at[pl.ds(off, BPW)], idxc_v.at[k], sem_i.at[k])

  def wait_idx(k):
    pltpu.make_async_copy(row_hbm.at[pl.ds(base, BPW)], idxr_v.at[k],
                          sem_i.at[k]).wait()
    pltpu.make_async_copy(col_hbm.at[pl.ds(base, BPW)], idxc_v.at[k],
                          sem_i.at[k]).wait()

  def fire_gather(p, k):
    pltpu.async_copy(x_hbm.at[idxc_v.at[k]], rows_v.at[p], sem_g.at[p])

  def wait_gather(p, k):
    pltpu.make_async_copy(x_hbm.at[idxc_v.at[k]], rows_v.at[p],
                          sem_g.at[p]).wait()

  def fire_scatter(p, k):
    pltpu.async_copy(rows_v.at[p], agg_sh.at[idxr_v.at[k]], sem_s.at[p],
                     add=True)

  def wait_scatter(p, k):
    pltpu.make_async_copy(rows_v.at[p], agg_sh.at[idxr_v.at[k]],
                          sem_s.at[p]).wait()

  pltpu.sync_copy(init_hbm.at[pl.ds(c * NP + zbase, RPS)],
                  agg_sh.at[pl.ds(zbase, RPS)])
  plsc.subcore_barrier()

  for k in range(4):
    fire_idx(k, k)
  for p in range(2):
    wait_idx(p)
    fire_gather(p, p)

  @pl.loop(0, NBLK_H, step=NIDXB)
  def _blocks(b):
    for q in range(NIDXB):
      blk = b + q
      p = q % NROWB
      wait_gather(p, q)
      fire_scatter(p, q)

      @pl.when(blk + 2 < NBLK_H)
      def _next_gather():
        wait_scatter(p, q)
        wait_idx((q + 2) % NIDXB)
        fire_gather(p, (q + 2) % NIDXB)

      @pl.when(blk + 4 < NBLK_H)
      def _next_idx():
        fire_idx((q + 4) % NIDXB, blk + 4)

  wait_scatter(0, 0)
  wait_scatter(1, 1)

  plsc.subcore_barrier()
  pltpu.sync_copy(agg_sh.at[pl.ds(zbase, RPS)],
                  out_hbm.at[pl.ds(c * NP + zbase, RPS)])


_agg_pass = functools.partial(
    pl.kernel,
    out_type=jax.ShapeDtypeStruct((NC * NP, D), jnp.float32),
    mesh=plsc.VectorSubcoreMesh(core_axis_name="c", subcore_axis_name="s"),
    scratch_types=[
        pltpu.VMEM((NIDXB, BPW), jnp.int32),
        pltpu.VMEM((NIDXB, BPW), jnp.int32),
        pltpu.VMEM((NROWB, BPW, D), jnp.float32),
        pltpu.VMEM_SHARED((NP, D), jnp.float32),
        pltpu.SemaphoreType.DMA((NIDXB,)),
        pltpu.SemaphoreType.DMA((NROWB,)),
        pltpu.SemaphoreType.DMA((NROWB,)),
    ],
)(_agg_body)


def _deg_body(row_hbm, zeros_hbm, ones_hbm, out_hbm,
              idxr_v, ones_v, deg_sh, sem_i, sem_s):
  c = lax.axis_index("c")
  s = lax.axis_index("s")
  w = s * NC + c
  base = w * EPW
  zbase = s * RPS

  def fire_idx(k, blk):
    pltpu.async_copy(row_hbm.at[pl.ds(base + blk * BPW, BPW)],
                     idxr_v.at[k], sem_i.at[k])

  def wait_idx(k):
    pltpu.make_async_copy(row_hbm.at[pl.ds(base, BPW)], idxr_v.at[k],
                          sem_i.at[k]).wait()

  def fire_scatter(k):
    pltpu.async_copy(ones_v, deg_sh.at[idxr_v.at[k]], sem_s.at[k], add=True)

  def wait_scatter(k):
    pltpu.make_async_copy(ones_v, deg_sh.at[idxr_v.at[k]],
                          sem_s.at[k]).wait()

  pltpu.sync_copy(ones_hbm, ones_v)
  pltpu.sync_copy(zeros_hbm.at[pl.ds(zbase, RPS)],
                  deg_sh.at[pl.ds(zbase, RPS)])
  plsc.subcore_barrier()

  for k in range(4):
    fire_idx(k, k)

  @pl.loop(0, NBLK, step=NIDXB)
  def _blocks(b):
    for q in range(NIDXB):
      blk = b + q
      wait_idx(q)
      fire_scatter(q)

      @pl.when(blk >= 4)
      def _drain():
        wait_scatter((q + 4) % NIDXB)

      @pl.when(blk + 4 < NBLK)
      def _next_idx():
        fire_idx((q + 4) % NIDXB, blk + 4)

  for k in range(4, NIDXB):
    wait_scatter(k)

  plsc.subcore_barrier()
  pltpu.sync_copy(deg_sh.at[pl.ds(zbase, RPS)],
                  out_hbm.at[pl.ds(c * NP + zbase, RPS)])


_deg_pass = functools.partial(
    pl.kernel,
    out_type=jax.ShapeDtypeStruct((NC * NP, D), jnp.float32),
    mesh=plsc.VectorSubcoreMesh(core_axis_name="c", subcore_axis_name="s"),
    scratch_types=[
        pltpu.VMEM((NIDXB, BPW), jnp.int32),
        pltpu.VMEM((BPW, D), jnp.float32),
        pltpu.VMEM_SHARED((NP, D), jnp.float32),
        pltpu.SemaphoreType.DMA((NIDXB,)),
        pltpu.SemaphoreType.DMA((NIDXB,)),
    ],
)(_deg_body)


ROWS_TC = 1000
NTCBLK = N // ROWS_TC


def _mlp_body(x_ref, agg2_ref, deg2_ref, w1x, w1a, b1, w2, b2, w3, b3, w4,
              b4, out_ref):
  deg = deg2_ref[0, :, 0:1] + deg2_ref[1, :, 0:1]
  deg = jnp.maximum(deg, 1.0)
  agg = (agg2_ref[0] + agg2_ref[1]) / deg
  h = jnp.dot(x_ref[...], w1x[...], preferred_element_type=jnp.float32)
  h += jnp.dot(agg, w1a[...], preferred_element_type=jnp.float32)
  h = jnp.maximum(h + b1[...], 0.0)
  h = jnp.maximum(
      jnp.dot(h, w2[...], preferred_element_type=jnp.float32) + b2[...], 0.0)
  h = jnp.maximum(
      jnp.dot(h, w3[...], preferred_element_type=jnp.float32) + b3[...], 0.0)
  out_ref[...] = (
      jnp.dot(h, w4[...], preferred_element_type=jnp.float32) + b4[...])


def _full_spec(shape):
  return pl.BlockSpec(shape, lambda i: tuple(0 for _ in shape))


_mlp = pl.pallas_call(
    _mlp_body,
    grid=(NTCBLK,),
    in_specs=[
        pl.BlockSpec((ROWS_TC, D), lambda i: (i, 0)),
        pl.BlockSpec((NC, ROWS_TC, D), lambda i: (0, i, 0)),
        pl.BlockSpec((NC, ROWS_TC, D), lambda i: (0, i, 0)),
        _full_spec((D, HID)),
        _full_spec((D, HID)),
        _full_spec((1, HID)),
        _full_spec((HID, HID)),
        _full_spec((1, HID)),
        _full_spec((HID, HID)),
        _full_spec((1, HID)),
        _full_spec((HID, D)),
        _full_spec((1, D)),
    ],
    out_specs=pl.BlockSpec((ROWS_TC, D), lambda i: (i, 0)),
    out_shape=jax.ShapeDtypeStruct((N, D), jnp.float32),
)


@jax.jit
def kernel(x, edge_index, W1, b1, W2, b2, W3, b3, W4, b4):
  row = edge_index[0].astype(jnp.int32)
  col = edge_index[1].astype(jnp.int32)
  pad = EPAD - E
  row_p = jnp.concatenate([row, jnp.full((pad,), N, jnp.int32)])
  col_p = jnp.concatenate([col, jnp.zeros((pad,), jnp.int32)])
  zeros_slab = jnp.zeros((NC * NP, D), jnp.float32)

  part1 = _agg_pass(x, row_p[:EH], col_p[:EH], zeros_slab)
  part2 = _agg_pass(x, row_p[EH:], col_p[EH:], part1)
  deg128 = _deg_pass(row_p, zeros_slab, jnp.ones((BPW, D), jnp.float32))

  agg2 = part2.reshape(NC, NP, D)
  deg2 = deg128.reshape(NC, NP, D)

  w1t = W1.T
  return _mlp(x, agg2, deg2, w1t[:D], w1t[D:], b1.reshape(1, HID),
              W2.T, b2.reshape(1, HID), W3.T, b3.reshape(1, HID),
              W4.T, b4.reshape(1, D))
